# baseline (speedup 1.0000x reference)
import functools

import jax
import jax.numpy as jnp
from jax.experimental import pallas as pl
from jax.experimental.pallas import tpu as pltpu

_ALPHA = 0.3
_CONFIDENCE = 2.0 / 3.0
_EPS2 = 1e-24

_BS = 256
_NCACHE = 17
_NI = 2
_NO = 2


def _pick(sims, attrs_blk, iota):
    s = jnp.where(iota < 5, sims, -jnp.inf)
    m = jnp.max(s)
    idx = jnp.min(jnp.where(s >= m, iota, 8))
    onehot = (iota == idx).astype(jnp.float32)
    return jnp.sum(onehot * attrs_blk, axis=0, keepdims=True)


def _alpha(j, s_total):
    row = (j * _BS + jax.lax.broadcasted_iota(jnp.int32, (_BS, 1), 0)).astype(
        jnp.float32
    )
    return (_ALPHA * _CONFIDENCE) * (0.5 + 0.5 * (row / s_total))


def _manual_kernel(
    h_ref, attrs_ref, out_ref, ibuf, obuf, stash, isem, osem, *, nb, s_total
):
    bs = _BS

    def in_copy(blk, slot):
        return pltpu.make_async_copy(
            h_ref.at[pl.ds(blk * bs, bs), :],
            ibuf.at[pl.ds(slot * bs, bs), :],
            isem.at[slot],
        )

    def out_copy(blk, slot):
        return pltpu.make_async_copy(
            obuf.at[pl.ds(slot * bs, bs), :],
            out_ref.at[pl.ds(blk * bs, bs), :],
            osem.at[slot],
        )

    for k in range(_NI):
        in_copy(k, k).start()

    def body_a(i, acc):
        slot = jax.lax.rem(i, _NI)
        in_copy(i, slot).wait()
        h = ibuf[pl.ds(slot * bs, bs), :]
        acc = acc + jnp.sum(h, axis=0, keepdims=True)

        @pl.when(i < _NCACHE)
        def _():
            rn2 = jnp.sum(h * h, axis=1, keepdims=True)
            inv = _alpha(i, s_total) * jax.lax.rsqrt(jnp.maximum(rn2, _EPS2))
            stash[pl.ds(i * bs, bs), :] = (h * (1.0 - inv)).astype(jnp.bfloat16)

        @pl.when(i + _NI < nb)
        def _():
            in_copy(i + _NI, slot).start()

        return acc

    acc = jax.lax.fori_loop(
        0, nb, body_a, jnp.zeros((1, attrs_ref.shape[1]), jnp.float32)
    )

    attrs = attrs_ref[...]
    sims = jnp.sum(acc * attrs, axis=1, keepdims=True)
    iota = jax.lax.broadcasted_iota(jnp.int32, (8, 1), 0)
    sel = _pick(sims[0:8], attrs[0:8], iota) + _pick(sims[8:16], attrs[8:16], iota)
    comb_raw = 0.5 * sel
    n = jnp.sqrt(jnp.sum(comb_raw * comb_raw))
    comb = comb_raw / jnp.maximum(n, 1e-12)

    for k in range(_NI):
        if _NCACHE + k < nb:
            in_copy(_NCACHE + k, (_NCACHE + k) % _NI).start()

    def body_b(j, carry):
        so = jax.lax.rem(j, _NO)

        @pl.when(j >= _NO)
        def _():
            out_copy(j - _NO, so).wait()

        @pl.when(j < _NCACHE)
        def _():
            scaled = stash[pl.ds(j * bs, bs), :].astype(jnp.float32)
            obuf[pl.ds(so * bs, bs), :] = scaled + _alpha(j, s_total) * comb

        @pl.when(j >= _NCACHE)
        def _():
            slot = jax.lax.rem(j, _NI)
            in_copy(j, slot).wait()
            h = ibuf[pl.ds(slot * bs, bs), :]
            rn2 = jnp.sum(h * h, axis=1, keepdims=True)
            a = _alpha(j, s_total)
            inv = a * jax.lax.rsqrt(jnp.maximum(rn2, _EPS2))
            obuf[pl.ds(so * bs, bs), :] = h * (1.0 - inv) + a * comb

            @pl.when(j + _NI < nb)
            def _():
                in_copy(j + _NI, slot).start()

        out_copy(j, so).start()
        return carry

    jax.lax.fori_loop(0, nb, body_b, 0)

    for k in range(_NO):
        blk = nb - _NO + k
        if blk >= 0:
            out_copy(blk, blk % _NO).wait()


def kernel(hidden_states, attr_implication, attr_modus_ponens):
    B, S, D = hidden_states.shape
    h = hidden_states.reshape(S, D)
    attrs = (
        jnp.zeros((16, D), jnp.float32)
        .at[0:5].set(attr_implication)
        .at[8:13].set(attr_modus_ponens)
    )
    nb = S // _BS

    out = pl.pallas_call(
        functools.partial(_manual_kernel, nb=nb, s_total=float(S)),
        in_specs=[
            pl.BlockSpec(memory_space=pltpu.MemorySpace.HBM),
            pl.BlockSpec((16, D), lambda: (0, 0)),
        ],
        out_specs=pl.BlockSpec(memory_space=pltpu.MemorySpace.HBM),
        out_shape=jax.ShapeDtypeStruct((S, D), jnp.float32),
        scratch_shapes=[
            pltpu.VMEM((_NI * _BS, D), jnp.float32),
            pltpu.VMEM((_NO * _BS, D), jnp.float32),
            pltpu.VMEM((_NCACHE * _BS, D), jnp.bfloat16),
            pltpu.SemaphoreType.DMA((_NI,)),
            pltpu.SemaphoreType.DMA((_NO,)),
        ],
    )(h, attrs)
    return out.reshape(B, S, D)


# speedup vs baseline: 1.0951x; 1.0951x over previous
import functools

import jax
import jax.numpy as jnp
from jax.experimental import pallas as pl
from jax.experimental.pallas import tpu as pltpu

_ALPHA = 0.3
_CONFIDENCE = 2.0 / 3.0
_EPS2 = 1e-24

_BS = 256
_NCACHE = 16
_NI = 3
_NO = 2


def _pick(sims, attrs_blk, iota):
    s = jnp.where(iota < 5, sims, -jnp.inf)
    m = jnp.max(s)
    idx = jnp.min(jnp.where(s >= m, iota, 8))
    onehot = (iota == idx).astype(jnp.float32)
    return jnp.sum(onehot * attrs_blk, axis=0, keepdims=True)


def _alpha(j, s_total, off, n):
    row = (j * _BS + off + jax.lax.broadcasted_iota(jnp.int32, (n, 1), 0)).astype(
        jnp.float32
    )
    return (_ALPHA * _CONFIDENCE) * (0.5 + 0.5 * (row / s_total))


def _manual_kernel(
    h_ref, attrs_ref, out_ref, ibuf, obuf, stash, isem, osem, *, nb, s_total
):
    bs = _BS

    def in_copy(blk, slot):
        return pltpu.make_async_copy(
            h_ref.at[pl.ds(blk * bs, bs), :],
            ibuf.at[pl.ds(slot * bs, bs), :],
            isem.at[slot],
        )

    def out_copy(blk, slot):
        return pltpu.make_async_copy(
            obuf.at[pl.ds(slot * bs, bs), :],
            out_ref.at[pl.ds(blk * bs, bs), :],
            osem.at[slot],
        )

    for k in range(_NI):
        in_copy(k, k).start()

    def body_a(i, acc):
        slot = jax.lax.rem(i, _NI)
        in_copy(i, slot).wait()
        h = ibuf[pl.ds(slot * bs, bs), :]
        acc = acc + jnp.sum(h, axis=0, keepdims=True)

        @pl.when(i < _NCACHE)
        def _():
            for c in range(2):
                hc = ibuf[pl.ds(slot * bs + c * (bs // 2), bs // 2), :]
                rn2 = jnp.sum(hc * hc, axis=1, keepdims=True)
                inv = _alpha(i, s_total, c * (bs // 2), bs // 2) * jax.lax.rsqrt(
                    jnp.maximum(rn2, _EPS2)
                )
                stash[pl.ds(i * bs + c * (bs // 2), bs // 2), :] = (
                    hc * (1.0 - inv)
                ).astype(jnp.bfloat16)

        @pl.when(i + _NI < nb)
        def _():
            in_copy(i + _NI, slot).start()

        return acc

    acc = jax.lax.fori_loop(
        0, nb, body_a, jnp.zeros((1, attrs_ref.shape[1]), jnp.float32)
    )

    attrs = attrs_ref[...]
    sims = jnp.sum(acc * attrs, axis=1, keepdims=True)
    iota = jax.lax.broadcasted_iota(jnp.int32, (8, 1), 0)
    sel = _pick(sims[0:8], attrs[0:8], iota) + _pick(sims[8:16], attrs[8:16], iota)
    comb_raw = 0.5 * sel
    n = jnp.sqrt(jnp.sum(comb_raw * comb_raw))
    comb = comb_raw / jnp.maximum(n, 1e-12)

    for k in range(_NI):
        if _NCACHE + k < nb:
            in_copy(_NCACHE + k, (_NCACHE + k) % _NI).start()

    def body_b(j, carry):
        so = jax.lax.rem(j, _NO)

        @pl.when(j >= _NO)
        def _():
            out_copy(j - _NO, so).wait()

        @pl.when(j < _NCACHE)
        def _():
            for c in range(2):
                scaled = stash[pl.ds(j * bs + c * (bs // 2), bs // 2), :].astype(
                    jnp.float32
                )
                obuf[pl.ds(so * bs + c * (bs // 2), bs // 2), :] = (
                    scaled + _alpha(j, s_total, c * (bs // 2), bs // 2) * comb
                )

        @pl.when(j >= _NCACHE)
        def _():
            slot = jax.lax.rem(j, _NI)
            in_copy(j, slot).wait()
            for c in range(2):
                hc = ibuf[pl.ds(slot * bs + c * (bs // 2), bs // 2), :]
                rn2 = jnp.sum(hc * hc, axis=1, keepdims=True)
                a = _alpha(j, s_total, c * (bs // 2), bs // 2)
                inv = a * jax.lax.rsqrt(jnp.maximum(rn2, _EPS2))
                obuf[pl.ds(so * bs + c * (bs // 2), bs // 2), :] = (
                    hc * (1.0 - inv) + a * comb
                )

            @pl.when(j + _NI < nb)
            def _():
                in_copy(j + _NI, slot).start()

        out_copy(j, so).start()
        return carry

    jax.lax.fori_loop(0, nb, body_b, 0)

    for k in range(_NO):
        blk = nb - _NO + k
        if blk >= 0:
            out_copy(blk, blk % _NO).wait()


def kernel(hidden_states, attr_implication, attr_modus_ponens):
    B, S, D = hidden_states.shape
    h = hidden_states.reshape(S, D)
    attrs = (
        jnp.zeros((16, D), jnp.float32)
        .at[0:5].set(attr_implication)
        .at[8:13].set(attr_modus_ponens)
    )
    nb = S // _BS

    out = pl.pallas_call(
        functools.partial(_manual_kernel, nb=nb, s_total=float(S)),
        in_specs=[
            pl.BlockSpec(memory_space=pltpu.MemorySpace.HBM),
            pl.BlockSpec((16, D), lambda: (0, 0)),
        ],
        out_specs=pl.BlockSpec(memory_space=pltpu.MemorySpace.HBM),
        out_shape=jax.ShapeDtypeStruct((S, D), jnp.float32),
        scratch_shapes=[
            pltpu.VMEM((_NI * _BS, D), jnp.float32),
            pltpu.VMEM((_NO * _BS, D), jnp.float32),
            pltpu.VMEM((_NCACHE * _BS, D), jnp.bfloat16),
            pltpu.SemaphoreType.DMA((_NI,)),
            pltpu.SemaphoreType.DMA((_NO,)),
        ],
    )(h, attrs)
    return out.reshape(B, S, D)


# speedup vs baseline: 1.1077x; 1.0115x over previous
import functools

import jax
import jax.numpy as jnp
from jax.experimental import pallas as pl
from jax.experimental.pallas import tpu as pltpu

_ALPHA = 0.3
_CONFIDENCE = 2.0 / 3.0
_EPS2 = 1e-24

_BS = 256
_NCACHE = 17
_NI = 3
_NO = 2


def _pick(sims, attrs_blk, iota):
    s = jnp.where(iota < 5, sims, -jnp.inf)
    m = jnp.max(s)
    idx = jnp.min(jnp.where(s >= m, iota, 8))
    onehot = (iota == idx).astype(jnp.float32)
    return jnp.sum(onehot * attrs_blk, axis=0, keepdims=True)


def _alpha(j, s_total, off, n):
    row = (j * _BS + off + jax.lax.broadcasted_iota(jnp.int32, (n, 1), 0)).astype(
        jnp.float32
    )
    return (_ALPHA * _CONFIDENCE) * (0.5 + 0.5 * (row / s_total))


def _manual_kernel(
    h_ref, attrs_ref, out_ref, ibuf, obuf, stash, isem, osem, *, nb, s_total
):
    bs = _BS

    def in_copy(blk, slot):
        return pltpu.make_async_copy(
            h_ref.at[pl.ds(blk * bs, bs), :],
            ibuf.at[pl.ds(slot * bs, bs), :],
            isem.at[slot],
        )

    def out_copy(blk, slot):
        return pltpu.make_async_copy(
            obuf.at[pl.ds(slot * bs, bs), :],
            out_ref.at[pl.ds(blk * bs, bs), :],
            osem.at[slot],
        )

    for k in range(_NI):
        in_copy(k, k).start()

    def body_a(i, acc):
        slot = jax.lax.rem(i, _NI)
        in_copy(i, slot).wait()
        h = ibuf[pl.ds(slot * bs, bs), :]
        acc = acc + jnp.sum(h, axis=0, keepdims=True)

        @pl.when(i < _NCACHE)
        def _():
            for c in range(2):
                hc = ibuf[pl.ds(slot * bs + c * (bs // 2), bs // 2), :]
                rn2 = jnp.sum(hc * hc, axis=1, keepdims=True)
                inv = _alpha(i, s_total, c * (bs // 2), bs // 2) * jax.lax.rsqrt(
                    jnp.maximum(rn2, _EPS2)
                )
                stash[pl.ds(i * bs + c * (bs // 2), bs // 2), :] = (
                    hc * (1.0 - inv)
                ).astype(jnp.bfloat16)

        @pl.when(i + _NI < nb)
        def _():
            in_copy(i + _NI, slot).start()

        return acc

    acc = jax.lax.fori_loop(
        0, nb, body_a, jnp.zeros((1, attrs_ref.shape[1]), jnp.float32)
    )

    attrs = attrs_ref[...]
    sims = jnp.sum(acc * attrs, axis=1, keepdims=True)
    iota = jax.lax.broadcasted_iota(jnp.int32, (8, 1), 0)
    sel = _pick(sims[0:8], attrs[0:8], iota) + _pick(sims[8:16], attrs[8:16], iota)
    comb_raw = 0.5 * sel
    n = jnp.sqrt(jnp.sum(comb_raw * comb_raw))
    comb = comb_raw / jnp.maximum(n, 1e-12)

    for k in range(_NI):
        if _NCACHE + k < nb:
            in_copy(_NCACHE + k, (_NCACHE + k) % _NI).start()

    def body_b(j, carry):
        so = jax.lax.rem(j, _NO)

        @pl.when(j >= _NO)
        def _():
            out_copy(j - _NO, so).wait()

        @pl.when(j < _NCACHE)
        def _():
            for c in range(2):
                scaled = stash[pl.ds(j * bs + c * (bs // 2), bs // 2), :].astype(
                    jnp.float32
                )
                obuf[pl.ds(so * bs + c * (bs // 2), bs // 2), :] = (
                    scaled + _alpha(j, s_total, c * (bs // 2), bs // 2) * comb
                )

        @pl.when(j >= _NCACHE)
        def _():
            slot = jax.lax.rem(j, _NI)
            in_copy(j, slot).wait()
            for c in range(2):
                hc = ibuf[pl.ds(slot * bs + c * (bs // 2), bs // 2), :]
                rn2 = jnp.sum(hc * hc, axis=1, keepdims=True)
                a = _alpha(j, s_total, c * (bs // 2), bs // 2)
                inv = a * jax.lax.rsqrt(jnp.maximum(rn2, _EPS2))
                obuf[pl.ds(so * bs + c * (bs // 2), bs // 2), :] = (
                    hc * (1.0 - inv) + a * comb
                )

            @pl.when(j + _NI < nb)
            def _():
                in_copy(j + _NI, slot).start()

        out_copy(j, so).start()
        return carry

    jax.lax.fori_loop(0, nb, body_b, 0)

    for k in range(_NO):
        blk = nb - _NO + k
        if blk >= 0:
            out_copy(blk, blk % _NO).wait()


def kernel(hidden_states, attr_implication, attr_modus_ponens):
    B, S, D = hidden_states.shape
    h = hidden_states.reshape(S, D)
    attrs = (
        jnp.zeros((16, D), jnp.float32)
        .at[0:5].set(attr_implication)
        .at[8:13].set(attr_modus_ponens)
    )
    nb = S // _BS

    out = pl.pallas_call(
        functools.partial(_manual_kernel, nb=nb, s_total=float(S)),
        in_specs=[
            pl.BlockSpec(memory_space=pltpu.MemorySpace.HBM),
            pl.BlockSpec((16, D), lambda: (0, 0)),
        ],
        out_specs=pl.BlockSpec(memory_space=pltpu.MemorySpace.HBM),
        out_shape=jax.ShapeDtypeStruct((S, D), jnp.float32),
        scratch_shapes=[
            pltpu.VMEM((_NI * _BS, D), jnp.float32),
            pltpu.VMEM((_NO * _BS, D), jnp.float32),
            pltpu.VMEM((_NCACHE * _BS, D), jnp.bfloat16),
            pltpu.SemaphoreType.DMA((_NI,)),
            pltpu.SemaphoreType.DMA((_NO,)),
        ],
    )(h, attrs)
    return out.reshape(B, S, D)
